# baseline (speedup 1.0000x reference)
import functools

import jax
import jax.numpy as jnp
from jax.experimental import pallas as pl
from jax.experimental.pallas import tpu as pltpu


def _conv_stats_kernel(x_ref, w_ref, y_ref, stats_ref, *, H, W):
    x4 = x_ref[0]
    cin = x4.shape[0]
    hw = H * W
    lo = W + 1
    x = x4.reshape(cin, hw).astype(jnp.bfloat16)
    xe = jnp.pad(x, ((0, 0), (lo, lo)))
    scol = jnp.mod(
        jax.lax.broadcasted_iota(jnp.int32, (1, hw + 2 * lo), 1) - lo, W
    )
    x_l = jnp.where(scol == W - 1, jnp.bfloat16(0), xe)
    x_r = jnp.where(scol == 0, jnp.bfloat16(0), xe)
    srcs = (x_l, xe, x_r)
    taps = []
    for kh in range(3):
        for kw in range(3):
            d = (kh - 1) * W + (kw - 1)
            taps.append(srcs[kw][:, lo + d:lo + d + hw])
    patches = jnp.concatenate(taps, axis=0)
    y = jnp.dot(w_ref[...], patches, preferred_element_type=jnp.float32)
    s = jnp.sum(y, axis=1, keepdims=True)
    ss = jnp.sum(y * y, axis=1, keepdims=True)
    stats_ref[0] = jnp.concatenate([s, ss], axis=1)
    y_ref[0] = y.astype(y_ref.dtype)


def _bn_relu_kernel(y_ref, sc_ref, sh_ref, o_ref, *, H, W):
    cout = y_ref.shape[1]
    y3 = y_ref[0].reshape(cout, H, W).astype(jnp.float32)
    sc = sc_ref[...].reshape(cout, 1, 1)
    sh = sh_ref[...].reshape(cout, 1, 1)
    o_ref[0] = jnp.maximum(y3 * sc + sh, 0.0).astype(o_ref.dtype)


@jax.jit
def kernel(x_nchw, w_oihw, gamma, beta):
    eps = 1e-5
    N, Cin, H, W = x_nchw.shape
    Cout, _, KH, KW = w_oihw.shape
    HW = H * W
    K = KH * KW * Cin

    wflat = (
        jnp.transpose(w_oihw, (0, 2, 3, 1)).reshape(Cout, K).astype(jnp.bfloat16)
    )

    conv_body = functools.partial(_conv_stats_kernel, H=H, W=W)
    flops1 = 2 * N * HW * K * Cout
    bytes1 = x_nchw.size * 4 + wflat.size * 2 + N * Cout * HW * 2 + N * Cout * 2 * 4
    y, stats = pl.pallas_call(
        conv_body,
        out_shape=(
            jax.ShapeDtypeStruct((N, Cout, HW), jnp.bfloat16),
            jax.ShapeDtypeStruct((N, Cout, 2), jnp.float32),
        ),
        grid=(N,),
        in_specs=[
            pl.BlockSpec((1, Cin, H, W), lambda n: (n, 0, 0, 0)),
            pl.BlockSpec((Cout, K), lambda n: (0, 0)),
        ],
        out_specs=(
            pl.BlockSpec((1, Cout, HW), lambda n: (n, 0, 0)),
            pl.BlockSpec((1, Cout, 2), lambda n: (n, 0, 0)),
        ),
        compiler_params=pltpu.CompilerParams(
            dimension_semantics=("parallel",),
            vmem_limit_bytes=48 * 1024 * 1024,
        ),
        cost_estimate=pl.CostEstimate(
            flops=flops1, transcendentals=0, bytes_accessed=bytes1
        ),
    )(x_nchw, wflat)

    totals = jnp.sum(stats, axis=0)
    count = N * HW
    mean = totals[:, 0] / count
    var = jnp.maximum(totals[:, 1] / count - mean * mean, 0.0)
    scale = gamma.astype(jnp.float32) * jax.lax.rsqrt(var + eps)
    shift = beta.astype(jnp.float32) - mean * scale

    bn_body = functools.partial(_bn_relu_kernel, H=H, W=W)
    bytes2 = N * Cout * HW * (2 + 4) + 2 * Cout * 4
    out = pl.pallas_call(
        bn_body,
        out_shape=jax.ShapeDtypeStruct((N, Cout, H, W), x_nchw.dtype),
        grid=(N,),
        in_specs=[
            pl.BlockSpec((1, Cout, HW), lambda n: (n, 0, 0)),
            pl.BlockSpec((Cout, 1), lambda n: (0, 0)),
            pl.BlockSpec((Cout, 1), lambda n: (0, 0)),
        ],
        out_specs=pl.BlockSpec((1, Cout, H, W), lambda n: (n, 0, 0, 0)),
        compiler_params=pltpu.CompilerParams(
            dimension_semantics=("parallel",),
            vmem_limit_bytes=32 * 1024 * 1024,
        ),
        cost_estimate=pl.CostEstimate(
            flops=2 * N * Cout * HW, transcendentals=0, bytes_accessed=bytes2
        ),
    )(y, scale.reshape(Cout, 1), shift.reshape(Cout, 1))

    return out


# speedup vs baseline: 1.8144x; 1.8144x over previous
import functools

import jax
import jax.numpy as jnp
from jax.experimental import pallas as pl
from jax.experimental.pallas import tpu as pltpu


def _conv_stats_kernel(x_ref, w_ref, y_ref, stats_ref, *, H, W):
    x = x_ref[0].astype(jnp.bfloat16)
    hw, cin = x.shape
    xe = jnp.pad(x, ((W + 1, W + 1), (0, 0)))
    g = jax.lax.broadcasted_iota(jnp.int32, (hw + 2 * W + 2, 1), 0)
    gm = jnp.mod(g, W)
    x_0 = jnp.where(gm == 0, jnp.bfloat16(0), xe)
    x_2 = jnp.where(gm == 1, jnp.bfloat16(0), xe)
    span = hw + 2 * W
    patches = jnp.concatenate(
        [x_0[0:span], xe[1:span + 1], x_2[2:span + 2]], axis=1
    )
    y = jnp.dot(patches[0:hw], w_ref[0],
                preferred_element_type=jnp.float32)
    y += jnp.dot(patches[W:W + hw], w_ref[1],
                 preferred_element_type=jnp.float32)
    y += jnp.dot(patches[2 * W:2 * W + hw], w_ref[2],
                 preferred_element_type=jnp.float32)
    s = jnp.sum(y, axis=0, keepdims=True)
    ss = jnp.sum(y * y, axis=0, keepdims=True)
    stats_ref[0] = jnp.concatenate([s, ss], axis=0)
    y_ref[0] = y.astype(y_ref.dtype)


def _bn_relu_kernel(y_ref, sc_ref, sh_ref, o_ref):
    y = y_ref[0].astype(jnp.float32)
    o_ref[0] = jnp.maximum(y * sc_ref[...] + sh_ref[...], 0.0).astype(o_ref.dtype)


@jax.jit
def kernel(x_nchw, w_oihw, gamma, beta):
    eps = 1e-5
    N, Cin, H, W = x_nchw.shape
    Cout, _, KH, KW = w_oihw.shape
    HW = H * W

    x_hwc = jnp.transpose(x_nchw, (0, 2, 3, 1)).reshape(N, HW, Cin)
    w_k = (
        jnp.transpose(w_oihw, (2, 3, 1, 0))
        .reshape(KH, KW * Cin, Cout)
        .astype(jnp.bfloat16)
    )

    conv_body = functools.partial(_conv_stats_kernel, H=H, W=W)
    flops1 = 2 * N * HW * KH * KW * Cin * Cout
    bytes1 = x_hwc.size * 4 + w_k.size * 2 + N * HW * Cout * 2 + N * 2 * Cout * 4
    y, stats = pl.pallas_call(
        conv_body,
        out_shape=(
            jax.ShapeDtypeStruct((N, HW, Cout), jnp.bfloat16),
            jax.ShapeDtypeStruct((N, 2, Cout), jnp.float32),
        ),
        grid=(N,),
        in_specs=[
            pl.BlockSpec((1, HW, Cin), lambda n: (n, 0, 0)),
            pl.BlockSpec((KH, KW * Cin, Cout), lambda n: (0, 0, 0)),
        ],
        out_specs=(
            pl.BlockSpec((1, HW, Cout), lambda n: (n, 0, 0)),
            pl.BlockSpec((1, 2, Cout), lambda n: (n, 0, 0)),
        ),
        compiler_params=pltpu.CompilerParams(
            dimension_semantics=("parallel",),
            vmem_limit_bytes=48 * 1024 * 1024,
        ),
        cost_estimate=pl.CostEstimate(
            flops=flops1, transcendentals=0, bytes_accessed=bytes1
        ),
    )(x_hwc, w_k)

    totals = jnp.sum(stats, axis=0)
    count = N * HW
    mean = totals[0] / count
    var = jnp.maximum(totals[1] / count - mean * mean, 0.0)
    scale = gamma.astype(jnp.float32) * jax.lax.rsqrt(var + eps)
    shift = beta.astype(jnp.float32) - mean * scale

    bytes2 = N * HW * Cout * (2 + 4) + 2 * Cout * 4
    out = pl.pallas_call(
        _bn_relu_kernel,
        out_shape=jax.ShapeDtypeStruct((N, HW, Cout), x_nchw.dtype),
        grid=(N,),
        in_specs=[
            pl.BlockSpec((1, HW, Cout), lambda n: (n, 0, 0)),
            pl.BlockSpec((1, Cout), lambda n: (0, 0)),
            pl.BlockSpec((1, Cout), lambda n: (0, 0)),
        ],
        out_specs=pl.BlockSpec((1, HW, Cout), lambda n: (n, 0, 0)),
        compiler_params=pltpu.CompilerParams(
            dimension_semantics=("parallel",),
            vmem_limit_bytes=32 * 1024 * 1024,
        ),
        cost_estimate=pl.CostEstimate(
            flops=2 * N * HW * Cout, transcendentals=0, bytes_accessed=bytes2
        ),
    )(y, scale.reshape(1, Cout), shift.reshape(1, Cout))

    return jnp.transpose(out.reshape(N, H, W, Cout), (0, 3, 1, 2))
